# baseline (speedup 1.0000x reference)
import functools

import jax
import jax.numpy as jnp
from jax import lax
from jax.experimental import pallas as pl
from jax.experimental.pallas import tpu as pltpu
from jax.experimental.pallas import tpu_sc as plsc

NUM_HEADS = 12
AREA = 576 * 576
TABLE_N = 2209 * NUM_HEADS
NC, NS, L = 2, 16, 16
NW = NC * NS
CHUNK = AREA // NW
BATCH = 16
HG = 4


def _sc_gather_bias(table_flat, idx_flat):
    mesh = plsc.VectorSubcoreMesh(core_axis_name="c", subcore_axis_name="s")
    NQ = 4
    QTR = CHUNK // NQ

    @functools.partial(
        pl.kernel,
        out_type=jax.ShapeDtypeStruct((NUM_HEADS, AREA), jnp.float32),
        mesh=mesh,
        scratch_types=[
            pltpu.VMEM((TABLE_N,), jnp.float32),
            pltpu.VMEM((CHUNK,), jnp.int32),
            pltpu.VMEM((2, NUM_HEADS, QTR), jnp.float32),
            pltpu.SemaphoreType.DMA,
            pltpu.SemaphoreType.DMA,
        ],
        compiler_params=pltpu.CompilerParams(
            needs_layout_passes=False, use_tc_tiling_on_sc=False
        ),
        name="bias_gather",
    )
    def k(table_hbm, idx_hbm, out_hbm, tab_v, idx_v, rows_v, sem, dsem):
        wid = lax.axis_index("s") * NC + lax.axis_index("c")
        base = wid * CHUNK
        tab_cp = pltpu.async_copy(table_hbm, tab_v, sem)
        idx_cp = pltpu.async_copy(idx_hbm.at[pl.ds(base, CHUNK)], idx_v, sem)
        tab_cp.wait()
        idx_cp.wait()

        def gather_quarter(q, pipelined):
            off = q * QTR
            slot = q % 2

            def step(i):
                iv = idx_v[pl.ds(off + i * L, L)]
                g0 = iv * NUM_HEADS
                for h in range(NUM_HEADS):
                    rows_v[slot, h, pl.ds(i * L, L)] = plsc.load_gather(
                        tab_v, [g0 + h]
                    )

            if pipelined:
                plsc.parallel_loop(0, QTR // L, unroll=4)(step)
            else:
                def body(i, c):
                    step(i)
                    return c
                lax.fori_loop(0, QTR // L, body, 0, unroll=2)

        def drain_quarter(q):
            off = q * QTR
            slot = q % 2
            return [
                pltpu.async_copy(
                    rows_v.at[slot, h], out_hbm.at[h, pl.ds(base + off, QTR)], dsem
                )
                for h in range(NUM_HEADS)
            ]

        gather_quarter(0, pipelined=True)
        gather_quarter(1, pipelined=True)
        d0 = drain_quarter(0)
        for cp in d0:
            cp.wait()
        gather_quarter(2, pipelined=True)
        d1 = drain_quarter(1)
        for cp in d1:
            cp.wait()
        gather_quarter(3, pipelined=False)
        d2 = drain_quarter(2)
        d3 = drain_quarter(3)
        for cp in d2 + d3:
            cp.wait()

    return k(table_flat, idx_flat)


def _tc_add(attn, bias3):
    def body(attn_ref, bias_ref, out_ref):
        out_ref[...] = attn_ref[...] + bias_ref[...]

    return pl.pallas_call(
        body,
        grid=(NUM_HEADS // HG, BATCH),
        in_specs=[
            pl.BlockSpec((1, HG, 576, 576), lambda h, b: (b, h, 0, 0)),
            pl.BlockSpec((HG, 576, 576), lambda h, b: (h, 0, 0)),
        ],
        out_specs=pl.BlockSpec((1, HG, 576, 576), lambda h, b: (b, h, 0, 0)),
        out_shape=jax.ShapeDtypeStruct(attn.shape, attn.dtype),
    )(attn, bias3)


def kernel(attn, rel_pos_bias_table, rel_pos_index):
    table_flat = rel_pos_bias_table.reshape(TABLE_N)
    idx_flat = rel_pos_index.reshape(AREA).astype(jnp.int32)
    bias = _sc_gather_bias(table_flat, idx_flat)
    bias3 = bias.reshape(NUM_HEADS, 576, 576)
    return _tc_add(attn, bias3)


# speedup vs baseline: 1.0276x; 1.0276x over previous
import functools

import jax
import jax.numpy as jnp
from jax import lax
from jax.experimental import pallas as pl
from jax.experimental.pallas import tpu as pltpu
from jax.experimental.pallas import tpu_sc as plsc

NUM_HEADS = 12
AREA = 576 * 576
TABLE_N = 2209 * NUM_HEADS
NC, NS, L = 2, 16, 16
NW = NC * NS
CHUNK = AREA // NW
BATCH = 16
HG = 6


def _sc_gather_bias(table_flat, idx_flat):
    mesh = plsc.VectorSubcoreMesh(core_axis_name="c", subcore_axis_name="s")
    NQ = 6
    NBUF = 3
    QTR = CHUNK // NQ

    @functools.partial(
        pl.kernel,
        out_type=jax.ShapeDtypeStruct((NUM_HEADS, AREA), jnp.float32),
        mesh=mesh,
        scratch_types=[
            pltpu.VMEM((TABLE_N,), jnp.float32),
            pltpu.VMEM((CHUNK,), jnp.int32),
            pltpu.VMEM((NBUF, NUM_HEADS, QTR), jnp.float32),
            pltpu.SemaphoreType.DMA,
            pltpu.SemaphoreType.DMA,
        ],
        compiler_params=pltpu.CompilerParams(
            needs_layout_passes=False, use_tc_tiling_on_sc=False
        ),
        name="bias_gather",
    )
    def k(table_hbm, idx_hbm, out_hbm, tab_v, idx_v, rows_v, sem, dsem):
        wid = lax.axis_index("s") * NC + lax.axis_index("c")
        base = wid * CHUNK
        tab_cp = pltpu.async_copy(table_hbm, tab_v, sem)
        idx_cp = pltpu.async_copy(idx_hbm.at[pl.ds(base, CHUNK)], idx_v, sem)
        tab_cp.wait()
        idx_cp.wait()

        def gather_quarter(q, pipelined):
            off = q * QTR
            slot = q % NBUF

            def step(i):
                iv = idx_v[pl.ds(off + i * L, L)]
                g0 = iv * NUM_HEADS
                for h in range(NUM_HEADS):
                    rows_v[slot, h, pl.ds(i * L, L)] = plsc.load_gather(
                        tab_v, [g0 + h]
                    )

            if pipelined:
                plsc.parallel_loop(0, QTR // L, unroll=4)(step)
            else:
                def body(i, c):
                    step(i)
                    return c
                lax.fori_loop(0, QTR // L, body, 0, unroll=2)

        def drain_quarter(q):
            off = q * QTR
            slot = q % NBUF
            return [
                pltpu.async_copy(
                    rows_v.at[slot, h], out_hbm.at[h, pl.ds(base + off, QTR)], dsem
                )
                for h in range(NUM_HEADS)
            ]

        drains = {}
        for q in range(NQ):
            if q >= NBUF:
                for cp in drains[q - NBUF]:
                    cp.wait()
            gather_quarter(q, pipelined=(q < NQ - 1))
            if q >= 1:
                drains[q - 1] = drain_quarter(q - 1)
        drains[NQ - 1] = drain_quarter(NQ - 1)
        for q in (NQ - NBUF, NQ - 2, NQ - 1):
            for cp in drains[q]:
                cp.wait()

    return k(table_flat, idx_flat)


def _tc_add(attn, bias3):
    def body(attn_ref, bias_ref, out_ref):
        out_ref[...] = attn_ref[...] + bias_ref[...]

    return pl.pallas_call(
        body,
        grid=(NUM_HEADS // HG, BATCH),
        in_specs=[
            pl.BlockSpec((1, HG, 576, 576), lambda h, b: (b, h, 0, 0)),
            pl.BlockSpec((HG, 576, 576), lambda h, b: (h, 0, 0)),
        ],
        out_specs=pl.BlockSpec((1, HG, 576, 576), lambda h, b: (b, h, 0, 0)),
        out_shape=jax.ShapeDtypeStruct(attn.shape, attn.dtype),
    )(attn, bias3)


def kernel(attn, rel_pos_bias_table, rel_pos_index):
    table_flat = rel_pos_bias_table.reshape(TABLE_N)
    idx_flat = rel_pos_index.reshape(AREA).astype(jnp.int32)
    bias = _sc_gather_bias(table_flat, idx_flat)
    bias3 = bias.reshape(NUM_HEADS, 576, 576)
    return _tc_add(attn, bias3)
